# baseline (speedup 1.0000x reference)
import functools

import jax
import jax.numpy as jnp
from jax import lax
from jax.experimental import pallas as pl
from jax.experimental.pallas import tpu as pltpu
from jax.experimental.pallas import tpu_sc as plsc

DIM = 64
BATCH = 16384
NC = 2
NS = 16
NW = NC * NS
RPW = BATCH // NW
CHUNK = 128
NCH = RPW // CHUNK
LANES = 16
VPR = DIM // LANES


def _neg_sqrt(x):
    xi = plsc.bitcast(x, jnp.int32)
    yi = jnp.int32(0x5F3759DF) - lax.shift_right_logical(xi, 1)
    y = plsc.bitcast(yi, jnp.float32)
    half_x = 0.5 * x
    for _ in range(3):
        y = y * (1.5 - half_x * y * y)
    return -(x * y)


def _sc_body(h_hbm, r_hbm, t_hbm, e_hbm, rel_hbm, out_hbm,
             idx_h, idx_r, idx_t, eh, er, et, ssq, out_v, sem):
    wid = lax.axis_index("s") * NC + lax.axis_index("c")
    base = wid * RPW

    iota = lax.broadcasted_iota(jnp.int32, (LANES,), 0)

    for c in range(NCH):
        off = base + c * CHUNK
        pltpu.sync_copy(h_hbm.at[pl.ds(off, CHUNK)], idx_h)
        pltpu.sync_copy(r_hbm.at[pl.ds(off, CHUNK)], idx_r)
        pltpu.sync_copy(t_hbm.at[pl.ds(off, CHUNK)], idx_t)
        cp1 = pltpu.async_copy(e_hbm.at[idx_h], eh, sem)
        cp2 = pltpu.async_copy(rel_hbm.at[idx_r], er, sem)
        cp3 = pltpu.async_copy(e_hbm.at[idx_t], et, sem)
        cp1.wait()
        cp2.wait()
        cp3.wait()

        def row_body(i, _):
            s = None
            for v in range(VPR):
                d = (eh[i, pl.ds(v * LANES, LANES)]
                     + er[i, pl.ds(v * LANES, LANES)]
                     - et[i, pl.ds(v * LANES, LANES)])
                sq = d * d
                s = sq if s is None else s + sq
            ssq[pl.ds(i * LANES, LANES)] = s
            return 0

        lax.fori_loop(0, CHUNK, row_body, 0, unroll=4)

        def grp_body(g, _):
            gbase = g * (LANES * LANES)
            acc = plsc.load_gather(ssq, [gbase + iota * LANES])
            for j in range(1, LANES):
                acc = acc + plsc.load_gather(ssq, [gbase + iota * LANES + j])
            out_v[pl.ds(c * CHUNK + g * LANES, LANES)] = _neg_sqrt(acc)
            return 0

        lax.fori_loop(0, CHUNK // LANES, grp_body, 0)

    pltpu.sync_copy(out_v, out_hbm.at[pl.ds(base, RPW)])


@jax.jit
def _transe_scores(h, r, t, E, R):
    mesh = plsc.VectorSubcoreMesh(core_axis_name="c", subcore_axis_name="s")
    kern = functools.partial(
        pl.kernel,
        mesh=mesh,
        compiler_params=pltpu.CompilerParams(
            needs_layout_passes=False, use_tc_tiling_on_sc=False),
        out_type=jax.ShapeDtypeStruct((BATCH,), jnp.float32),
        scratch_types=[
            pltpu.VMEM((CHUNK,), jnp.int32),
            pltpu.VMEM((CHUNK,), jnp.int32),
            pltpu.VMEM((CHUNK,), jnp.int32),
            pltpu.VMEM((CHUNK, DIM), jnp.float32),
            pltpu.VMEM((CHUNK, DIM), jnp.float32),
            pltpu.VMEM((CHUNK, DIM), jnp.float32),
            pltpu.VMEM((CHUNK * LANES,), jnp.float32),
            pltpu.VMEM((RPW,), jnp.float32),
            pltpu.SemaphoreType.DMA,
        ],
    )(_sc_body)
    return kern(h, r, t, E, R)


def kernel(triples, E, R):
    h = triples[:, 0]
    r = triples[:, 1]
    t = triples[:, 2]
    return _transe_scores(h, r, t, E, R)


# speedup vs baseline: 1.5587x; 1.5587x over previous
import functools

import jax
import jax.numpy as jnp
from jax import lax
from jax.experimental import pallas as pl
from jax.experimental.pallas import tpu as pltpu
from jax.experimental.pallas import tpu_sc as plsc

DIM = 64
BATCH = 16384
NC = 2
NS = 16
NW = NC * NS
RPW = BATCH // NW
CHUNK = 128
NCH = RPW // CHUNK
LANES = 16
VPR = DIM // LANES


def _neg_sqrt(x):
    xi = plsc.bitcast(x, jnp.int32)
    yi = jnp.int32(0x5F3759DF) - lax.shift_right_logical(xi, 1)
    y = plsc.bitcast(yi, jnp.float32)
    half_x = 0.5 * x
    for _ in range(3):
        y = y * (1.5 - half_x * y * y)
    return -(x * y)


def _sc_body(h_hbm, r_hbm, t_hbm, e_hbm, rel_hbm, out_hbm,
             idx_h, idx_r, idx_t, eh, er, et, ssq, out_v, sem):
    wid = lax.axis_index("s") * NC + lax.axis_index("c")
    base = wid * RPW

    iota = lax.broadcasted_iota(jnp.int32, (LANES,), 0)

    for c in range(NCH):
        off = base + c * CHUNK
        pltpu.sync_copy(h_hbm.at[pl.ds(off, CHUNK)], idx_h)
        pltpu.sync_copy(r_hbm.at[pl.ds(off, CHUNK)], idx_r)
        pltpu.sync_copy(t_hbm.at[pl.ds(off, CHUNK)], idx_t)

        def fire_body(g, _):
            gb = g * LANES
            hv = idx_h[pl.ds(gb, LANES)]
            rv = idx_r[pl.ds(gb, LANES)]
            tv = idx_t[pl.ds(gb, LANES)]
            for j in range(LANES):
                pltpu.async_copy(e_hbm.at[pl.ds(hv[j], 1)],
                                 eh.at[pl.ds(gb + j, 1)], sem)
                pltpu.async_copy(rel_hbm.at[pl.ds(rv[j], 1)],
                                 er.at[pl.ds(gb + j, 1)], sem)
                pltpu.async_copy(e_hbm.at[pl.ds(tv[j], 1)],
                                 et.at[pl.ds(gb + j, 1)], sem)
            return 0

        lax.fori_loop(0, CHUNK // LANES, fire_body, 0)

        def drain_body(i, _):
            pltpu.make_async_copy(e_hbm.at[pl.ds(0, 1)], eh.at[pl.ds(i, 1)],
                                  sem).wait()
            pltpu.make_async_copy(rel_hbm.at[pl.ds(0, 1)], er.at[pl.ds(i, 1)],
                                  sem).wait()
            pltpu.make_async_copy(e_hbm.at[pl.ds(0, 1)], et.at[pl.ds(i, 1)],
                                  sem).wait()
            return 0

        lax.fori_loop(0, CHUNK, drain_body, 0)

        def row_body(i, _):
            s = None
            for v in range(VPR):
                d = (eh[i, pl.ds(v * LANES, LANES)]
                     + er[i, pl.ds(v * LANES, LANES)]
                     - et[i, pl.ds(v * LANES, LANES)])
                sq = d * d
                s = sq if s is None else s + sq
            ssq[pl.ds(i * LANES, LANES)] = s
            return 0

        lax.fori_loop(0, CHUNK, row_body, 0, unroll=4)

        def grp_body(g, _):
            gbase = g * (LANES * LANES)
            acc = plsc.load_gather(ssq, [gbase + iota * LANES])
            for j in range(1, LANES):
                acc = acc + plsc.load_gather(ssq, [gbase + iota * LANES + j])
            out_v[pl.ds(c * CHUNK + g * LANES, LANES)] = _neg_sqrt(acc)
            return 0

        lax.fori_loop(0, CHUNK // LANES, grp_body, 0)

    pltpu.sync_copy(out_v, out_hbm.at[pl.ds(base, RPW)])


@jax.jit
def _transe_scores(h, r, t, E, R):
    mesh = plsc.VectorSubcoreMesh(core_axis_name="c", subcore_axis_name="s")
    kern = functools.partial(
        pl.kernel,
        mesh=mesh,
        compiler_params=pltpu.CompilerParams(needs_layout_passes=False),
        out_type=jax.ShapeDtypeStruct((BATCH,), jnp.float32),
        scratch_types=[
            pltpu.VMEM((CHUNK,), jnp.int32),
            pltpu.VMEM((CHUNK,), jnp.int32),
            pltpu.VMEM((CHUNK,), jnp.int32),
            pltpu.VMEM((CHUNK, DIM), jnp.float32),
            pltpu.VMEM((CHUNK, DIM), jnp.float32),
            pltpu.VMEM((CHUNK, DIM), jnp.float32),
            pltpu.VMEM((CHUNK * LANES,), jnp.float32),
            pltpu.VMEM((RPW,), jnp.float32),
            pltpu.SemaphoreType.DMA,
        ],
    )(_sc_body)
    return kern(h, r, t, E, R)


def kernel(triples, E, R):
    h = triples[:, 0]
    r = triples[:, 1]
    t = triples[:, 2]
    return _transe_scores(h, r, t, E, R)


# speedup vs baseline: 2.0287x; 1.3016x over previous
import functools

import jax
import jax.numpy as jnp
from jax import lax
from jax.experimental import pallas as pl
from jax.experimental.pallas import tpu as pltpu
from jax.experimental.pallas import tpu_sc as plsc

DIM = 64
BATCH = 16384
NC = 2
NS = 16
NW = NC * NS
LANES = 16
VPR = DIM // LANES

SUP = 512
NSUP_FULL = 1953
TAIL_LO = 999936
CAP_E = 6144
CAP_R = 4096
XE_ROWS = 2 * BATCH + LANES
XR_ROWS = BATCH + LANES

RPW = BATCH // NW
CCH = 128


def _neg_sqrt(x):
    xi = plsc.bitcast(x, jnp.int32)
    yi = jnp.int32(0x5F3759DF) - lax.shift_right_logical(xi, 1)
    y = plsc.bitcast(yi, jnp.float32)
    half_x = 0.5 * x
    for _ in range(3):
        y = y * (1.5 - half_x * y * y)
    return -(x * y)


def _extract_body(h_hbm, r_hbm, t_hbm, et_hbm, rt_hbm,
                  etail_hbm, rtail_hbm, xe_hbm, xr_hbm,
                  idxbuf, he_idx, he_slot, hr_idx, hr_slot,
                  selcc, selslot, cpan, stage, etail, rtail, sem):
    wid = lax.axis_index("s") * NC + lax.axis_index("c")
    iota = lax.broadcasted_iota(jnp.int32, (LANES,), 0)
    zeros16 = iota * 0
    neg1 = zeros16 - 1

    def scan_list(list_hbm, slot_base, hidx, hslot, n0):
        def chunk_body(ci, n):
            pltpu.sync_copy(list_hbm.at[pl.ds(ci * 2048, 2048)], idxbuf)

            def vreg_body(q, n):
                iv = idxbuf[pl.ds(q * LANES, LANES)]
                sv = slot_base + ci * 2048 + q * LANES + iota
                mask = (lax.shift_right_logical(iv, 9) & 31) == wid
                plsc.store_compressed(hidx.at[pl.ds(n, LANES)], iv, mask=mask)
                plsc.store_compressed(hslot.at[pl.ds(n, LANES)], sv, mask=mask)
                k = plsc.all_reduce_population_count(mask)[0]
                return n + k

            return lax.fori_loop(0, 2048 // LANES, vreg_body, n)

        return lax.fori_loop(0, BATCH // 2048, chunk_body, n0)

    n_e = scan_list(h_hbm, 0, he_idx, he_slot, 0)
    n_e = scan_list(t_hbm, BATCH, he_idx, he_slot, n_e)
    n_r = scan_list(r_hbm, 0, hr_idx, hr_slot, 0)
    he_idx[pl.ds(n_e, LANES)] = neg1
    hr_idx[pl.ds(n_r, LANES)] = neg1

    pltpu.sync_copy(etail_hbm, etail)
    pltpu.sync_copy(rtail_hbm, rtail)

    def select_hits(hidx, hslot, nhits, su, base, dump_base):
        def sel_body(q, m):
            hv = hidx[pl.ds(q * LANES, LANES)]
            sv = hslot[pl.ds(q * LANES, LANES)]
            mask = lax.shift_right_logical(hv, 9) == su
            ccv = hv - base
            plsc.store_compressed(selcc.at[pl.ds(m, LANES)], ccv, mask=mask)
            plsc.store_compressed(selslot.at[pl.ds(m, LANES)], sv, mask=mask)
            return m + plsc.all_reduce_population_count(mask)[0]

        nq = lax.shift_right_logical(nhits + 15, 4)
        m = lax.fori_loop(0, nq, sel_body, 0)
        selcc[pl.ds(m, LANES)] = zeros16
        selslot[pl.ds(m, LANES)] = dump_base + iota
        return lax.shift_right_logical(m + 15, 4)

    def extract_tail(hidx, hslot, nhits, su, base, tailbuf, x_hbm,
                     dump_base):
        ngr = select_hits(hidx, hslot, nhits, su, base, dump_base)

        def grp(g, _):
            ccv = selcc[pl.ds(g * LANES, LANES)]
            slv = selslot[pl.ds(g * LANES, LANES)]
            for j in range(LANES):
                for v in range(VPR):
                    stage[j, pl.ds(v * LANES, LANES)] = (
                        tailbuf[ccv[j], pl.ds(v * LANES, LANES)])
            pltpu.async_copy(stage, x_hbm.at[slv], sem).wait()
            return 0

        lax.fori_loop(0, ngr, grp, 0)

    def extract(hidx, hslot, nhits, su, base, x_hbm, dump_base):
        ngr = select_hits(hidx, hslot, nhits, su, base, dump_base)

        def grp(g, _):
            ccv = selcc[pl.ds(g * LANES, LANES)]
            slv = selslot[pl.ds(g * LANES, LANES)]
            for j in range(LANES):
                ccsplat = zeros16 + ccv[j]
                for v in range(VPR):
                    g16 = plsc.load_gather(
                        cpan, [v * LANES + iota, ccsplat])
                    stage[j, pl.ds(v * LANES, LANES)] = g16
            pltpu.async_copy(stage, x_hbm.at[slv], sem).wait()
            return 0

        lax.fori_loop(0, ngr, grp, 0)

    def sup_body(s, _):
        su = wid + NW * s

        @pl.when(su < NSUP_FULL)
        def _full():
            off = pl.multiple_of(su * SUP, 128)
            pltpu.sync_copy(et_hbm.at[:, pl.ds(off, SUP)], cpan)
            extract(he_idx, he_slot, n_e, su, su * SUP, xe_hbm, 2 * BATCH)
            pltpu.sync_copy(rt_hbm.at[:, pl.ds(off, SUP)], cpan)
            extract(hr_idx, hr_slot, n_r, su, su * SUP, xr_hbm, BATCH)

        @pl.when(su == NSUP_FULL)
        def _tail():
            extract_tail(he_idx, he_slot, n_e, su, TAIL_LO, etail,
                         xe_hbm, 2 * BATCH)
            extract_tail(hr_idx, hr_slot, n_r, su, TAIL_LO, rtail,
                         xr_hbm, BATCH)

        return 0

    lax.fori_loop(0, 62, sup_body, 0)


def _compute_body(xe_hbm, xr_hbm, out_hbm, eh, er, et, ssq, out_v, sem):
    wid = lax.axis_index("s") * NC + lax.axis_index("c")
    base = wid * RPW
    iota = lax.broadcasted_iota(jnp.int32, (LANES,), 0)

    for c in range(RPW // CCH):
        off = base + c * CCH
        cp1 = pltpu.async_copy(xe_hbm.at[pl.ds(off, CCH)], eh, sem)
        cp2 = pltpu.async_copy(xr_hbm.at[pl.ds(off, CCH)], er, sem)
        cp3 = pltpu.async_copy(xe_hbm.at[pl.ds(BATCH + off, CCH)], et, sem)
        cp1.wait()
        cp2.wait()
        cp3.wait()

        def row_body(i, _):
            s = None
            for v in range(VPR):
                d = (eh[i, pl.ds(v * LANES, LANES)]
                     + er[i, pl.ds(v * LANES, LANES)]
                     - et[i, pl.ds(v * LANES, LANES)])
                sq = d * d
                s = sq if s is None else s + sq
            ssq[pl.ds(i * LANES, LANES)] = s
            return 0

        lax.fori_loop(0, CCH, row_body, 0, unroll=4)

        def grp_body(g, _):
            gbase = g * (LANES * LANES)
            acc = plsc.load_gather(ssq, [gbase + iota * LANES])
            for j in range(1, LANES):
                acc = acc + plsc.load_gather(ssq, [gbase + iota * LANES + j])
            out_v[pl.ds(c * CCH + g * LANES, LANES)] = _neg_sqrt(acc)
            return 0

        lax.fori_loop(0, CCH // LANES, grp_body, 0)

    pltpu.sync_copy(out_v, out_hbm.at[pl.ds(base, RPW)])


@jax.jit
def _transe_scores(triples, E, R):
    h = triples[:, 0]
    r = triples[:, 1]
    t = triples[:, 2]
    ET = E.T
    RT = R.T
    mesh = plsc.VectorSubcoreMesh(core_axis_name="c", subcore_axis_name="s")
    extract_k = functools.partial(
        pl.kernel,
        mesh=mesh,
        compiler_params=pltpu.CompilerParams(needs_layout_passes=False),
        out_type=[jax.ShapeDtypeStruct((XE_ROWS, 128), jnp.float32),
                  jax.ShapeDtypeStruct((XR_ROWS, 128), jnp.float32)],
        scratch_types=[
            pltpu.VMEM((2048,), jnp.int32),
            pltpu.VMEM((CAP_E + LANES,), jnp.int32),
            pltpu.VMEM((CAP_E + LANES,), jnp.int32),
            pltpu.VMEM((CAP_R + LANES,), jnp.int32),
            pltpu.VMEM((CAP_R + LANES,), jnp.int32),
            pltpu.VMEM((CAP_E + LANES,), jnp.int32),
            pltpu.VMEM((CAP_E + LANES,), jnp.int32),
            pltpu.VMEM((DIM, SUP), jnp.float32),
            pltpu.VMEM((LANES, 128), jnp.float32),
            pltpu.VMEM((DIM, 128), jnp.float32),
            pltpu.VMEM((DIM, 128), jnp.float32),
            pltpu.SemaphoreType.DMA,
        ],
    )(_extract_body)
    ntail = 1000000 - TAIL_LO
    etail_in = jnp.zeros((DIM, 128), jnp.float32).at[:ntail, :DIM].set(
        E[TAIL_LO:, :])
    rtail_in = jnp.zeros((DIM, 128), jnp.float32).at[:ntail, :DIM].set(
        R[TAIL_LO:, :])
    xe, xr = extract_k(h, r, t, ET, RT, etail_in, rtail_in)

    compute_k = functools.partial(
        pl.kernel,
        mesh=mesh,
        compiler_params=pltpu.CompilerParams(needs_layout_passes=False),
        out_type=jax.ShapeDtypeStruct((BATCH,), jnp.float32),
        scratch_types=[
            pltpu.VMEM((CCH, 128), jnp.float32),
            pltpu.VMEM((CCH, 128), jnp.float32),
            pltpu.VMEM((CCH, 128), jnp.float32),
            pltpu.VMEM((CCH * LANES,), jnp.float32),
            pltpu.VMEM((RPW,), jnp.float32),
            pltpu.SemaphoreType.DMA,
        ],
    )(_compute_body)
    return compute_k(xe, xr)


def kernel(triples, E, R):
    return _transe_scores(triples, E, R)


# speedup vs baseline: 2.2482x; 1.1082x over previous
import functools

import jax
import jax.numpy as jnp
from jax import lax
from jax.experimental import pallas as pl
from jax.experimental.pallas import tpu as pltpu
from jax.experimental.pallas import tpu_sc as plsc

DIM = 64
BATCH = 16384
NC = 2
NS = 16
NW = NC * NS
LANES = 16
VPR = DIM // LANES

SUP = 512
NSUP_FULL = 1953
TAIL_LO = 999936
CAP_E = 6144
CAP_R = 4096
XE_ROWS = 2 * BATCH + LANES
XR_ROWS = BATCH + LANES

RPW = BATCH // NW
CCH = 128


def _neg_sqrt(x):
    xi = plsc.bitcast(x, jnp.int32)
    yi = jnp.int32(0x5F3759DF) - lax.shift_right_logical(xi, 1)
    y = plsc.bitcast(yi, jnp.float32)
    half_x = 0.5 * x
    for _ in range(3):
        y = y * (1.5 - half_x * y * y)
    return -(x * y)


def _extract_body(h_hbm, r_hbm, t_hbm, et_hbm, rt_hbm,
                  etail_hbm, rtail_hbm, xe_hbm, xr_hbm,
                  idxbuf, he_idx, he_slot, hr_idx, hr_slot,
                  selcc, selslot, cpanE, cpanR, stage, etail, rtail,
                  sem, semE, semR):
    wid = lax.axis_index("s") * NC + lax.axis_index("c")
    iota = lax.broadcasted_iota(jnp.int32, (LANES,), 0)
    zeros16 = iota * 0
    neg1 = zeros16 - 1

    def scan_list(list_hbm, slot_base, hidx, hslot, n0):
        def chunk_body(ci, n):
            pltpu.sync_copy(list_hbm.at[pl.ds(ci * 2048, 2048)], idxbuf)

            def vreg_body(q, n):
                iv = idxbuf[pl.ds(q * LANES, LANES)]
                sv = slot_base + ci * 2048 + q * LANES + iota
                mask = (lax.shift_right_logical(iv, 9) & 31) == wid
                plsc.store_compressed(hidx.at[pl.ds(n, LANES)], iv, mask=mask)
                plsc.store_compressed(hslot.at[pl.ds(n, LANES)], sv, mask=mask)
                k = plsc.all_reduce_population_count(mask)[0]
                return n + k

            return lax.fori_loop(0, 2048 // LANES, vreg_body, n)

        return lax.fori_loop(0, BATCH // 2048, chunk_body, n0)

    n_e = scan_list(h_hbm, 0, he_idx, he_slot, 0)
    n_e = scan_list(t_hbm, BATCH, he_idx, he_slot, n_e)
    n_r = scan_list(r_hbm, 0, hr_idx, hr_slot, 0)
    he_idx[pl.ds(n_e, LANES)] = neg1
    hr_idx[pl.ds(n_r, LANES)] = neg1

    pltpu.sync_copy(etail_hbm, etail)
    pltpu.sync_copy(rtail_hbm, rtail)

    def select_hits(hidx, hslot, nhits, su, base, dump_base):
        def sel_body(q, m):
            hv = hidx[pl.ds(q * LANES, LANES)]
            sv = hslot[pl.ds(q * LANES, LANES)]
            mask = lax.shift_right_logical(hv, 9) == su
            ccv = hv - base
            plsc.store_compressed(selcc.at[pl.ds(m, LANES)], ccv, mask=mask)
            plsc.store_compressed(selslot.at[pl.ds(m, LANES)], sv, mask=mask)
            return m + plsc.all_reduce_population_count(mask)[0]

        nq = lax.shift_right_logical(nhits + 15, 4)
        m = lax.fori_loop(0, nq, sel_body, 0)
        selcc[pl.ds(m, LANES)] = zeros16
        selslot[pl.ds(m, LANES)] = dump_base + iota
        return lax.shift_right_logical(m + 15, 4)

    def extract_tail(hidx, hslot, nhits, su, base, tailbuf, x_hbm,
                     dump_base):
        ngr = select_hits(hidx, hslot, nhits, su, base, dump_base)

        def grp(g, _):
            ccv = selcc[pl.ds(g * LANES, LANES)]
            slv = selslot[pl.ds(g * LANES, LANES)]
            for j in range(LANES):
                for v in range(VPR):
                    stage[j, pl.ds(v * LANES, LANES)] = (
                        tailbuf[ccv[j], pl.ds(v * LANES, LANES)])
            pltpu.async_copy(stage, x_hbm.at[slv], sem).wait()
            return 0

        lax.fori_loop(0, ngr, grp, 0)

    def extract(hidx, hslot, nhits, su, base, pan, x_hbm, dump_base):
        ngr = select_hits(hidx, hslot, nhits, su, base, dump_base)

        def grp(g, _):
            ccv = selcc[pl.ds(g * LANES, LANES)]
            slv = selslot[pl.ds(g * LANES, LANES)]
            for j in range(LANES):
                ccsplat = zeros16 + ccv[j]
                for v in range(VPR):
                    g16 = plsc.load_gather(
                        pan, [v * LANES + iota, ccsplat])
                    stage[j, pl.ds(v * LANES, LANES)] = g16
            pltpu.async_copy(stage, x_hbm.at[slv], sem).wait()
            return 0

        lax.fori_loop(0, ngr, grp, 0)

    def clamp_off(s):
        su_f = jnp.minimum(wid + NW * s, NSUP_FULL - 1)
        return pl.multiple_of(su_f * SUP, 128)

    pltpu.async_copy(et_hbm.at[:, pl.ds(clamp_off(0), SUP)], cpanE, semE)

    def sup_body(s, _):
        su = wid + NW * s
        pltpu.make_async_copy(et_hbm.at[:, pl.ds(0, SUP)], cpanE,
                              semE).wait()
        pltpu.async_copy(rt_hbm.at[:, pl.ds(clamp_off(s), SUP)], cpanR,
                         semR)

        @pl.when(su < NSUP_FULL)
        def _full_e():
            extract(he_idx, he_slot, n_e, su, su * SUP, cpanE,
                    xe_hbm, 2 * BATCH)

        @pl.when(su == NSUP_FULL)
        def _tail_e():
            extract_tail(he_idx, he_slot, n_e, su, TAIL_LO, etail,
                         xe_hbm, 2 * BATCH)

        pltpu.make_async_copy(rt_hbm.at[:, pl.ds(0, SUP)], cpanR,
                              semR).wait()
        pltpu.async_copy(et_hbm.at[:, pl.ds(clamp_off(s + 1), SUP)], cpanE,
                         semE)

        @pl.when(su < NSUP_FULL)
        def _full_r():
            extract(hr_idx, hr_slot, n_r, su, su * SUP, cpanR,
                    xr_hbm, BATCH)

        @pl.when(su == NSUP_FULL)
        def _tail_r():
            extract_tail(hr_idx, hr_slot, n_r, su, TAIL_LO, rtail,
                         xr_hbm, BATCH)

        return 0

    lax.fori_loop(0, 62, sup_body, 0)
    pltpu.make_async_copy(et_hbm.at[:, pl.ds(0, SUP)], cpanE, semE).wait()


def _compute_body(xe_hbm, xr_hbm, out_hbm, eh, er, et, ssq, out_v, sem):
    wid = lax.axis_index("s") * NC + lax.axis_index("c")
    base = wid * RPW
    iota = lax.broadcasted_iota(jnp.int32, (LANES,), 0)

    for c in range(RPW // CCH):
        off = base + c * CCH
        cp1 = pltpu.async_copy(xe_hbm.at[pl.ds(off, CCH)], eh, sem)
        cp2 = pltpu.async_copy(xr_hbm.at[pl.ds(off, CCH)], er, sem)
        cp3 = pltpu.async_copy(xe_hbm.at[pl.ds(BATCH + off, CCH)], et, sem)
        cp1.wait()
        cp2.wait()
        cp3.wait()

        def row_body(i, _):
            s = None
            for v in range(VPR):
                d = (eh[i, pl.ds(v * LANES, LANES)]
                     + er[i, pl.ds(v * LANES, LANES)]
                     - et[i, pl.ds(v * LANES, LANES)])
                sq = d * d
                s = sq if s is None else s + sq
            ssq[pl.ds(i * LANES, LANES)] = s
            return 0

        lax.fori_loop(0, CCH, row_body, 0, unroll=4)

        def grp_body(g, _):
            gbase = g * (LANES * LANES)
            acc = plsc.load_gather(ssq, [gbase + iota * LANES])
            for j in range(1, LANES):
                acc = acc + plsc.load_gather(ssq, [gbase + iota * LANES + j])
            out_v[pl.ds(c * CCH + g * LANES, LANES)] = _neg_sqrt(acc)
            return 0

        lax.fori_loop(0, CCH // LANES, grp_body, 0)

    pltpu.sync_copy(out_v, out_hbm.at[pl.ds(base, RPW)])


@jax.jit
def _transe_scores(triples, E, R):
    h = triples[:, 0]
    r = triples[:, 1]
    t = triples[:, 2]
    ET = E.T
    RT = R.T
    mesh = plsc.VectorSubcoreMesh(core_axis_name="c", subcore_axis_name="s")
    extract_k = functools.partial(
        pl.kernel,
        mesh=mesh,
        compiler_params=pltpu.CompilerParams(needs_layout_passes=False),
        out_type=[jax.ShapeDtypeStruct((XE_ROWS, 128), jnp.float32),
                  jax.ShapeDtypeStruct((XR_ROWS, 128), jnp.float32)],
        scratch_types=[
            pltpu.VMEM((2048,), jnp.int32),
            pltpu.VMEM((CAP_E + LANES,), jnp.int32),
            pltpu.VMEM((CAP_E + LANES,), jnp.int32),
            pltpu.VMEM((CAP_R + LANES,), jnp.int32),
            pltpu.VMEM((CAP_R + LANES,), jnp.int32),
            pltpu.VMEM((CAP_E + LANES,), jnp.int32),
            pltpu.VMEM((CAP_E + LANES,), jnp.int32),
            pltpu.VMEM((DIM, SUP), jnp.float32),
            pltpu.VMEM((DIM, SUP), jnp.float32),
            pltpu.VMEM((LANES, 128), jnp.float32),
            pltpu.VMEM((DIM, 128), jnp.float32),
            pltpu.VMEM((DIM, 128), jnp.float32),
            pltpu.SemaphoreType.DMA,
            pltpu.SemaphoreType.DMA,
            pltpu.SemaphoreType.DMA,
        ],
    )(_extract_body)
    ntail = 1000000 - TAIL_LO
    etail_in = jnp.zeros((DIM, 128), jnp.float32).at[:ntail, :DIM].set(
        E[TAIL_LO:, :])
    rtail_in = jnp.zeros((DIM, 128), jnp.float32).at[:ntail, :DIM].set(
        R[TAIL_LO:, :])
    xe, xr = extract_k(h, r, t, ET, RT, etail_in, rtail_in)

    compute_k = functools.partial(
        pl.kernel,
        mesh=mesh,
        compiler_params=pltpu.CompilerParams(needs_layout_passes=False),
        out_type=jax.ShapeDtypeStruct((BATCH,), jnp.float32),
        scratch_types=[
            pltpu.VMEM((CCH, 128), jnp.float32),
            pltpu.VMEM((CCH, 128), jnp.float32),
            pltpu.VMEM((CCH, 128), jnp.float32),
            pltpu.VMEM((CCH * LANES,), jnp.float32),
            pltpu.VMEM((RPW,), jnp.float32),
            pltpu.SemaphoreType.DMA,
        ],
    )(_compute_body)
    return compute_k(xe, xr)


def kernel(triples, E, R):
    return _transe_scores(triples, E, R)


# speedup vs baseline: 2.2578x; 1.0043x over previous
import functools

import jax
import jax.numpy as jnp
from jax import lax
from jax.experimental import pallas as pl
from jax.experimental.pallas import tpu as pltpu
from jax.experimental.pallas import tpu_sc as plsc

DIM = 64
BATCH = 16384
NC = 2
NS = 16
NW = NC * NS
LANES = 16
VPR = DIM // LANES

SUP = 512
NSUP_FULL = 1953
TAIL_LO = 999936
CAP_E = 6144
CAP_R = 4096
XE_ROWS = 2 * BATCH + LANES
XR_ROWS = BATCH + LANES

RPW = BATCH // NW
CCH = 128


def _neg_sqrt(x):
    xi = plsc.bitcast(x, jnp.int32)
    yi = jnp.int32(0x5F3759DF) - lax.shift_right_logical(xi, 1)
    y = plsc.bitcast(yi, jnp.float32)
    half_x = 0.5 * x
    for _ in range(3):
        y = y * (1.5 - half_x * y * y)
    return -(x * y)


def _extract_body(h_hbm, r_hbm, t_hbm, et_hbm, rt_hbm,
                  etail_hbm, rtail_hbm, xe_hbm, xr_hbm,
                  idxbuf, he_idx, he_slot, hr_idx, hr_slot,
                  selcc, selslot, cpanE, cpanR, stage, etail, rtail,
                  sem, semE, semR):
    wid = lax.axis_index("s") * NC + lax.axis_index("c")
    iota = lax.broadcasted_iota(jnp.int32, (LANES,), 0)
    zeros16 = iota * 0
    neg1 = zeros16 - 1

    def scan_list(list_hbm, slot_base, hidx, hslot, n0):
        def chunk_body(ci, n):
            pltpu.sync_copy(list_hbm.at[pl.ds(ci * 2048, 2048)], idxbuf)

            def vreg_body(q, n):
                iv = idxbuf[pl.ds(q * LANES, LANES)]
                sv = slot_base + ci * 2048 + q * LANES + iota
                mask = (lax.shift_right_logical(iv, 9) & 31) == wid
                plsc.store_compressed(hidx.at[pl.ds(n, LANES)], iv, mask=mask)
                plsc.store_compressed(hslot.at[pl.ds(n, LANES)], sv, mask=mask)
                k = plsc.all_reduce_population_count(mask)[0]
                return n + k

            return lax.fori_loop(0, 2048 // LANES, vreg_body, n, unroll=4)

        return lax.fori_loop(0, BATCH // 2048, chunk_body, n0)

    n_e = scan_list(h_hbm, 0, he_idx, he_slot, 0)
    n_e = scan_list(t_hbm, BATCH, he_idx, he_slot, n_e)
    n_r = scan_list(r_hbm, 0, hr_idx, hr_slot, 0)
    he_idx[pl.ds(n_e, LANES)] = neg1
    hr_idx[pl.ds(n_r, LANES)] = neg1

    pltpu.sync_copy(etail_hbm, etail)
    pltpu.sync_copy(rtail_hbm, rtail)

    def select_hits(hidx, hslot, nhits, su, base, dump_base):
        def sel_body(q, m):
            hv = hidx[pl.ds(q * LANES, LANES)]
            sv = hslot[pl.ds(q * LANES, LANES)]
            mask = lax.shift_right_logical(hv, 9) == su
            ccv = hv - base
            plsc.store_compressed(selcc.at[pl.ds(m, LANES)], ccv, mask=mask)
            plsc.store_compressed(selslot.at[pl.ds(m, LANES)], sv, mask=mask)
            return m + plsc.all_reduce_population_count(mask)[0]

        nq = lax.shift_right_logical(nhits + 15, 4)
        m = lax.fori_loop(0, nq, sel_body, 0)
        selcc[pl.ds(m, LANES)] = zeros16
        selslot[pl.ds(m, LANES)] = dump_base + iota
        return lax.shift_right_logical(m + 15, 4)

    def extract_tail(hidx, hslot, nhits, su, base, tailbuf, x_hbm,
                     dump_base):
        ngr = select_hits(hidx, hslot, nhits, su, base, dump_base)

        def grp(g, _):
            ccv = selcc[pl.ds(g * LANES, LANES)]
            slv = selslot[pl.ds(g * LANES, LANES)]
            for j in range(LANES):
                for v in range(VPR):
                    stage[j, pl.ds(v * LANES, LANES)] = (
                        tailbuf[ccv[j], pl.ds(v * LANES, LANES)])
            pltpu.async_copy(stage, x_hbm.at[slv], sem).wait()
            return 0

        lax.fori_loop(0, ngr, grp, 0)

    def extract(hidx, hslot, nhits, su, base, pan, x_hbm, dump_base):
        ngr = select_hits(hidx, hslot, nhits, su, base, dump_base)

        def grp(g, _):
            ccv = selcc[pl.ds(g * LANES, LANES)]
            slv = selslot[pl.ds(g * LANES, LANES)]
            for j in range(LANES):
                ccsplat = zeros16 + ccv[j]
                for v in range(VPR):
                    g16 = plsc.load_gather(
                        pan, [v * LANES + iota, ccsplat])
                    stage[j, pl.ds(v * LANES, LANES)] = g16
            pltpu.async_copy(stage, x_hbm.at[slv], sem).wait()
            return 0

        lax.fori_loop(0, ngr, grp, 0)

    def clamp_off(s):
        su_f = jnp.minimum(wid + NW * s, NSUP_FULL - 1)
        return pl.multiple_of(su_f * SUP, 128)

    pltpu.async_copy(et_hbm.at[:, pl.ds(clamp_off(0), SUP)], cpanE, semE)

    def sup_body(s, _):
        su = wid + NW * s
        pltpu.make_async_copy(et_hbm.at[:, pl.ds(0, SUP)], cpanE,
                              semE).wait()
        pltpu.async_copy(rt_hbm.at[:, pl.ds(clamp_off(s), SUP)], cpanR,
                         semR)

        @pl.when(su < NSUP_FULL)
        def _full_e():
            extract(he_idx, he_slot, n_e, su, su * SUP, cpanE,
                    xe_hbm, 2 * BATCH)

        @pl.when(su == NSUP_FULL)
        def _tail_e():
            extract_tail(he_idx, he_slot, n_e, su, TAIL_LO, etail,
                         xe_hbm, 2 * BATCH)

        pltpu.make_async_copy(rt_hbm.at[:, pl.ds(0, SUP)], cpanR,
                              semR).wait()
        pltpu.async_copy(et_hbm.at[:, pl.ds(clamp_off(s + 1), SUP)], cpanE,
                         semE)

        @pl.when(su < NSUP_FULL)
        def _full_r():
            extract(hr_idx, hr_slot, n_r, su, su * SUP, cpanR,
                    xr_hbm, BATCH)

        @pl.when(su == NSUP_FULL)
        def _tail_r():
            extract_tail(hr_idx, hr_slot, n_r, su, TAIL_LO, rtail,
                         xr_hbm, BATCH)

        return 0

    lax.fori_loop(0, 62, sup_body, 0)
    pltpu.make_async_copy(et_hbm.at[:, pl.ds(0, SUP)], cpanE, semE).wait()


def _compute_body(xe_hbm, xr_hbm, out_hbm, eh, er, et, ssq, out_v, sem):
    wid = lax.axis_index("s") * NC + lax.axis_index("c")
    base = wid * RPW
    iota = lax.broadcasted_iota(jnp.int32, (LANES,), 0)

    for c in range(RPW // CCH):
        off = base + c * CCH
        cp1 = pltpu.async_copy(xe_hbm.at[pl.ds(off, CCH)], eh, sem)
        cp2 = pltpu.async_copy(xr_hbm.at[pl.ds(off, CCH)], er, sem)
        cp3 = pltpu.async_copy(xe_hbm.at[pl.ds(BATCH + off, CCH)], et, sem)
        cp1.wait()
        cp2.wait()
        cp3.wait()

        def row_body(i, _):
            s = None
            for v in range(VPR):
                d = (eh[i, pl.ds(v * LANES, LANES)]
                     + er[i, pl.ds(v * LANES, LANES)]
                     - et[i, pl.ds(v * LANES, LANES)])
                sq = d * d
                s = sq if s is None else s + sq
            ssq[pl.ds(i * LANES, LANES)] = s
            return 0

        lax.fori_loop(0, CCH, row_body, 0, unroll=4)

        def grp_body(g, _):
            gbase = g * (LANES * LANES)
            acc = plsc.load_gather(ssq, [gbase + iota * LANES])
            for j in range(1, LANES):
                acc = acc + plsc.load_gather(ssq, [gbase + iota * LANES + j])
            out_v[pl.ds(c * CCH + g * LANES, LANES)] = _neg_sqrt(acc)
            return 0

        lax.fori_loop(0, CCH // LANES, grp_body, 0)

    pltpu.sync_copy(out_v, out_hbm.at[pl.ds(base, RPW)])


@jax.jit
def _transe_scores(triples, E, R):
    h = triples[:, 0]
    r = triples[:, 1]
    t = triples[:, 2]
    ET = E.T
    RT = R.T
    mesh = plsc.VectorSubcoreMesh(core_axis_name="c", subcore_axis_name="s")
    extract_k = functools.partial(
        pl.kernel,
        mesh=mesh,
        compiler_params=pltpu.CompilerParams(needs_layout_passes=False),
        out_type=[jax.ShapeDtypeStruct((XE_ROWS, 128), jnp.float32),
                  jax.ShapeDtypeStruct((XR_ROWS, 128), jnp.float32)],
        scratch_types=[
            pltpu.VMEM((2048,), jnp.int32),
            pltpu.VMEM((CAP_E + LANES,), jnp.int32),
            pltpu.VMEM((CAP_E + LANES,), jnp.int32),
            pltpu.VMEM((CAP_R + LANES,), jnp.int32),
            pltpu.VMEM((CAP_R + LANES,), jnp.int32),
            pltpu.VMEM((CAP_E + LANES,), jnp.int32),
            pltpu.VMEM((CAP_E + LANES,), jnp.int32),
            pltpu.VMEM((DIM, SUP), jnp.float32),
            pltpu.VMEM((DIM, SUP), jnp.float32),
            pltpu.VMEM((LANES, 128), jnp.float32),
            pltpu.VMEM((DIM, 128), jnp.float32),
            pltpu.VMEM((DIM, 128), jnp.float32),
            pltpu.SemaphoreType.DMA,
            pltpu.SemaphoreType.DMA,
            pltpu.SemaphoreType.DMA,
        ],
    )(_extract_body)
    ntail = 1000000 - TAIL_LO
    etail_in = jnp.zeros((DIM, 128), jnp.float32).at[:ntail, :DIM].set(
        E[TAIL_LO:, :])
    rtail_in = jnp.zeros((DIM, 128), jnp.float32).at[:ntail, :DIM].set(
        R[TAIL_LO:, :])
    xe, xr = extract_k(h, r, t, ET, RT, etail_in, rtail_in)

    compute_k = functools.partial(
        pl.kernel,
        mesh=mesh,
        compiler_params=pltpu.CompilerParams(needs_layout_passes=False),
        out_type=jax.ShapeDtypeStruct((BATCH,), jnp.float32),
        scratch_types=[
            pltpu.VMEM((CCH, 128), jnp.float32),
            pltpu.VMEM((CCH, 128), jnp.float32),
            pltpu.VMEM((CCH, 128), jnp.float32),
            pltpu.VMEM((CCH * LANES,), jnp.float32),
            pltpu.VMEM((RPW,), jnp.float32),
            pltpu.SemaphoreType.DMA,
        ],
    )(_compute_body)
    return compute_k(xe, xr)


def kernel(triples, E, R):
    return _transe_scores(triples, E, R)
